# Initial kernel scaffold; baseline (speedup 1.0000x reference)
import jax
import jax.numpy as jnp
from jax.experimental import pallas as pl


def kernel(features, rois):
    raise NotImplementedError("write your pallas kernel here")


# speedup vs baseline: 12.7767x; 12.7767x over previous
import functools

import jax
import jax.numpy as jnp
from jax import lax
from jax.experimental import pallas as pl
from jax.experimental.pallas import tpu as pltpu
from jax.experimental.pallas import tpu_sc as plsc

OUT = 7
PTS = OUT * OUT
PPAD = 64
L = 16


@functools.lru_cache(maxsize=None)
def _build(B, C, H, W, N):
    NC = 2
    NTILE = NC * 16
    NGRP = NTILE // B
    GRP = C // NGRP
    NB = 25
    NBLK = N // NB
    RPAD = ((N * 4 + L) + 127) // 128 * 128
    ROW = GRP * PTS

    mesh = plsc.VectorSubcoreMesh(core_axis_name="c", subcore_axis_name="s")

    @functools.partial(
        pl.kernel,
        mesh=mesh,
        compiler_params=pltpu.CompilerParams(
            needs_layout_passes=False, use_tc_tiling_on_sc=False),
        out_type=jax.ShapeDtypeStruct((B * N, C * PTS), jnp.float32),
        scratch_types=[
            pltpu.VMEM((GRP * H * W,), jnp.float32),
            pltpu.VMEM((RPAD,), jnp.float32),
            pltpu.VMEM((PPAD,), jnp.float32),
            pltpu.VMEM((PPAD,), jnp.float32),
            pltpu.VMEM((NB, ROW), jnp.float32),
        ],
    )
    def k(feat_hbm, rois_hbm, ig_hbm, jg_hbm, out_hbm,
          slab, roi_v, ig_v, jg_v, stg):
        wid = lax.axis_index("s") * NC + lax.axis_index("c")
        b = wid // NGRP
        g = wid % NGRP
        pltpu.sync_copy(
            feat_hbm.at[pl.ds((b * C + g * GRP) * H * W, GRP * H * W)], slab)
        pltpu.sync_copy(rois_hbm.at[pl.ds(b * RPAD, RPAD)], roi_v)
        pltpu.sync_copy(ig_hbm, ig_v)
        pltpu.sync_copy(jg_hbm, jg_v)

        iota = lax.iota(jnp.int32, L)
        mask0 = iota == 0
        coffs = [jnp.full((L,), c * H * W, jnp.int32) for c in range(GRP)]

        def roi_body(r, blk):
            ri = blk * NB + r
            rv = plsc.load_gather(roi_v, [jnp.broadcast_to(ri * 4, (L,)) + iota])
            x1 = rv[0]
            y1 = rv[1]
            rw = jnp.maximum(rv[2] - x1, 1.0)
            rh = jnp.maximum(rv[3] - y1, 1.0)
            x1v = jnp.broadcast_to(x1, (L,))
            y1v = jnp.broadcast_to(y1, (L,))
            rwv = jnp.broadcast_to(rw, (L,))
            rhv = jnp.broadcast_to(rh, (L,))
            rvv = jnp.broadcast_to(r, (L,))
            for kk in range(PPAD // L):
                jgv = jg_v[pl.ds(kk * L, L)]
                igv = ig_v[pl.ds(kk * L, L)]
                px = x1v + jgv * rwv
                py = y1v + igv * rhv
                xi = jnp.clip(px.astype(jnp.int32), 0, W - 2)
                yi = jnp.clip(py.astype(jnp.int32), 0, H - 2)
                wx1 = px - xi.astype(jnp.float32)
                wy1 = py - yi.astype(jnp.float32)
                wx0 = 1.0 - wx1
                wy0 = 1.0 - wy1
                w00 = wy0 * wx0
                w01 = wy0 * wx1
                w10 = wy1 * wx0
                w11 = wy1 * wx1
                i00 = yi * W + xi
                i01 = i00 + 1
                i10 = i00 + W
                i11 = i10 + 1
                for c in range(GRP):
                    co = coffs[c]
                    v00 = plsc.load_gather(slab, [i00 + co])
                    v01 = plsc.load_gather(slab, [i01 + co])
                    v10 = plsc.load_gather(slab, [i10 + co])
                    v11 = plsc.load_gather(slab, [i11 + co])
                    acc = v00 * w00 + v01 * w01 + v10 * w10 + v11 * w11
                    if kk < 3:
                        col = iota + (c * PTS + kk * L)
                        plsc.store_scatter(stg, [rvv, col], acc)
                    else:
                        col = jnp.full((L,), c * PTS + 48, jnp.int32)
                        plsc.store_scatter(stg, [rvv, col], acc, mask=mask0)
            return blk

        def blk_body(blk, carry):
            lax.fori_loop(0, NB, roi_body, blk)
            pltpu.sync_copy(
                stg,
                out_hbm.at[pl.ds(b * N + blk * NB, NB), pl.ds(g * ROW, ROW)],
            )
            return carry

        lax.fori_loop(0, NBLK, blk_body, 0)

    return k


def kernel(features, rois):
    B, C, H, W = features.shape
    N = rois.shape[1]
    steps = jnp.linspace(0.0, 1.0, OUT)
    yg, xg = jnp.meshgrid(steps, steps, indexing="ij")
    pad = jnp.zeros((PPAD - PTS,), jnp.float32)
    ig = jnp.concatenate([yg.reshape(-1).astype(jnp.float32), pad])
    jg = jnp.concatenate([xg.reshape(-1).astype(jnp.float32), pad])
    fn = _build(B, C, H, W, N)
    rpad = ((N * 4 + L) + 127) // 128 * 128
    rois_flat = jnp.pad(
        rois.reshape(B, N * 4), ((0, 0), (0, rpad - N * 4))).reshape(-1)
    out = fn(features.reshape(B * C * H * W), rois_flat, ig, jg)
    return out.reshape(B * N, C, OUT, OUT)


# speedup vs baseline: 14.2419x; 1.1147x over previous
import functools

import jax
import jax.numpy as jnp
from jax import lax
from jax.experimental import pallas as pl
from jax.experimental.pallas import tpu as pltpu
from jax.experimental.pallas import tpu_sc as plsc

OUT = 7
PTS = OUT * OUT
PPAD = 64
L = 16


@functools.lru_cache(maxsize=None)
def _build(B, C, H, W, N):
    NC = 2
    NTILE = NC * 16
    NGRP = NTILE // B
    GRP = C // NGRP
    NB = 25
    NBLK = N // NB
    RPAD = ((N * 4 + L) + 127) // 128 * 128
    ROW = GRP * PTS

    mesh = plsc.VectorSubcoreMesh(core_axis_name="c", subcore_axis_name="s")

    @functools.partial(
        pl.kernel,
        mesh=mesh,
        compiler_params=pltpu.CompilerParams(
            needs_layout_passes=False, use_tc_tiling_on_sc=False),
        out_type=jax.ShapeDtypeStruct((B * N, C * PTS), jnp.float32),
        scratch_types=[
            pltpu.VMEM((GRP * H * W,), jnp.float32),
            pltpu.VMEM((RPAD,), jnp.float32),
            pltpu.VMEM((PPAD,), jnp.float32),
            pltpu.VMEM((PPAD,), jnp.float32),
            pltpu.VMEM((NB, ROW), jnp.float32),
            pltpu.VMEM((NB, ROW), jnp.float32),
            pltpu.SemaphoreType.DMA,
            pltpu.SemaphoreType.DMA,
        ],
    )
    def k(feat_hbm, rois_hbm, ig_hbm, jg_hbm, out_hbm,
          slab, roi_v, ig_v, jg_v, stg_a, stg_b, sem_a, sem_b):
        wid = lax.axis_index("s") * NC + lax.axis_index("c")
        b = wid // NGRP
        g = wid % NGRP
        pltpu.sync_copy(
            feat_hbm.at[pl.ds((b * C + g * GRP) * H * W, GRP * H * W)], slab)
        pltpu.sync_copy(rois_hbm.at[pl.ds(b * RPAD, RPAD)], roi_v)
        pltpu.sync_copy(ig_hbm, ig_v)
        pltpu.sync_copy(jg_hbm, jg_v)

        iota = lax.iota(jnp.int32, L)
        coffs = [jnp.full((L,), c * H * W, jnp.int32) for c in range(GRP)]

        def interp_store(buf, rvv, px, py, col_of_c, mask):
            xi = jnp.clip(px.astype(jnp.int32), 0, W - 2)
            yi = jnp.clip(py.astype(jnp.int32), 0, H - 2)
            wx1 = px - xi.astype(jnp.float32)
            wy1 = py - yi.astype(jnp.float32)
            wx0 = 1.0 - wx1
            wy0 = 1.0 - wy1
            w00 = wy0 * wx0
            w01 = wy0 * wx1
            w10 = wy1 * wx0
            w11 = wy1 * wx1
            i00 = yi * W + xi
            i01 = i00 + 1
            i10 = i00 + W
            i11 = i10 + 1
            for c in range(GRP):
                co = coffs[c]
                v00 = plsc.load_gather(slab, [i00 + co])
                v01 = plsc.load_gather(slab, [i01 + co])
                v10 = plsc.load_gather(slab, [i10 + co])
                v11 = plsc.load_gather(slab, [i11 + co])
                acc = v00 * w00 + v01 * w01 + v10 * w10 + v11 * w11
                plsc.store_scatter(buf, [rvv, col_of_c(c)], acc, mask=mask)

        def make_roi_body(buf):
            def roi_body(r, blk):
                ri = blk * NB + r
                rv = plsc.load_gather(
                    roi_v, [jnp.broadcast_to(ri * 4, (L,)) + iota])
                x1 = rv[0]
                y1 = rv[1]
                rw = jnp.maximum(rv[2] - x1, 1.0)
                rh = jnp.maximum(rv[3] - y1, 1.0)
                rvv = jnp.broadcast_to(r, (L,))
                for kk in range(3):
                    jgv = jg_v[pl.ds(kk * L, L)]
                    igv = ig_v[pl.ds(kk * L, L)]
                    px = jnp.broadcast_to(x1, (L,)) + jgv * rw
                    py = jnp.broadcast_to(y1, (L,)) + igv * rh
                    interp_store(buf, rvv, px, py,
                                 lambda c, kk=kk: iota + (c * PTS + kk * L),
                                 None)
                return blk
            return roi_body

        def last_point_pass(buf, blk, go, cnt):
            ridx4 = (jnp.broadcast_to(blk * NB + go, (L,)) + iota) * 4
            x1 = plsc.load_gather(roi_v, [ridx4])
            y1 = plsc.load_gather(roi_v, [ridx4 + 1])
            x2 = plsc.load_gather(roi_v, [ridx4 + 2])
            y2 = plsc.load_gather(roi_v, [ridx4 + 3])
            px = x1 + jnp.maximum(x2 - x1, 1.0)
            py = y1 + jnp.maximum(y2 - y1, 1.0)
            mask = (iota < cnt) if cnt < L else None
            rvv = jnp.broadcast_to(go, (L,)) + iota
            interp_store(buf, rvv, px, py,
                         lambda c: jnp.full((L,), c * PTS + 48, jnp.int32),
                         mask)

        def make_fill(buf):
            def fill(blk):
                lax.fori_loop(0, NB, make_roi_body(buf), blk)
                for go in range(0, NB, L):
                    last_point_pass(buf, blk, go, min(L, NB - go))
            return fill

        bufs = (stg_a, stg_b)
        sems = (sem_a, sem_b)
        fills = (make_fill(stg_a), make_fill(stg_b))

        def dst(blk):
            return out_hbm.at[pl.ds(b * N + blk * NB, NB),
                              pl.ds(g * ROW, ROW)]

        def pair_body(p, carry):
            for h in range(2):
                blk = p * 2 + h

                @pl.when(p > 0)
                def _wait():
                    pltpu.make_async_copy(bufs[h], dst(0), sems[h]).wait()

                fills[h](blk)
                pltpu.make_async_copy(bufs[h], dst(blk), sems[h]).start()
            return carry

        lax.fori_loop(0, NBLK // 2, pair_body, 0)
        for h in range(2):
            pltpu.make_async_copy(bufs[h], dst(0), sems[h]).wait()

    return k


def kernel(features, rois):
    B, C, H, W = features.shape
    N = rois.shape[1]
    steps = jnp.linspace(0.0, 1.0, OUT)
    yg, xg = jnp.meshgrid(steps, steps, indexing="ij")
    pad = jnp.zeros((PPAD - PTS,), jnp.float32)
    ig = jnp.concatenate([yg.reshape(-1).astype(jnp.float32), pad])
    jg = jnp.concatenate([xg.reshape(-1).astype(jnp.float32), pad])
    fn = _build(B, C, H, W, N)
    rpad = ((N * 4 + L) + 127) // 128 * 128
    rois_flat = jnp.pad(
        rois.reshape(B, N * 4), ((0, 0), (0, rpad - N * 4))).reshape(-1)
    out = fn(features.reshape(B * C * H * W), rois_flat, ig, jg)
    return out.reshape(B * N, C, OUT, OUT)
